# baseline (speedup 1.0000x reference)
import functools

import jax
import jax.numpy as jnp
from jax.experimental import pallas as pl
from jax.experimental.pallas import tpu as pltpu

B = 16
L = 2048
ES = 128
D = ES * 5
H = D * 2
LAYERS = 4
NEG = 0.01
RT = 512
NRT = L // RT


def _leaky(x):
    return jnp.maximum(x, NEG * x)


def _main_body(words_ref, action_ref, invlen_ref, emb_ref,
               pW1_ref, pb1_ref, pW2_ref, pb2_ref,
               gW1_ref, gb1_ref, gW2_ref, gb2_ref,
               cW_ref, cb_ref,
               out_ref, pooled_ref,
               h_s):
    b = pl.program_id(0)
    lyr = pl.program_id(1)

    @pl.when(lyr == 0)
    def _embed():
        for j in range(5):
            wj = words_ref[0, :, j:j + 1]
            oh = (wj == jax.lax.broadcasted_iota(jnp.int32, (L, 32), 1)
                  ).astype(jnp.float32)
            we = jnp.dot(oh, emb_ref[...], preferred_element_type=jnp.float32)
            aj = action_ref[b, j]
            oha = (aj == jax.lax.broadcasted_iota(jnp.int32, (1, 32), 1)
                   ).astype(jnp.float32)
            ae = jnp.dot(oha, emb_ref[...], preferred_element_type=jnp.float32)
            out_ref[0, :, ES * j:ES * (j + 1)] = we + ae

    gsum = jnp.zeros((1, D), jnp.float32)
    for r in range(NRT):
        xt = out_ref[0, r * RT:(r + 1) * RT, :].astype(jnp.bfloat16)
        h1 = _leaky(jnp.dot(xt, pW1_ref[0], preferred_element_type=jnp.float32)
                    + pb1_ref[0])
        ht = _leaky(jnp.dot(h1.astype(jnp.bfloat16), pW2_ref[0],
                            preferred_element_type=jnp.float32)
                    + pb2_ref[0])
        h_s[r * RT:(r + 1) * RT, :] = ht.astype(jnp.bfloat16)
        gsum = gsum + jnp.sum(ht, axis=0, keepdims=True)

    inv = invlen_ref[b]
    grp = gsum * inv
    g1 = _leaky(jnp.dot(grp, gW1_ref[0], preferred_element_type=jnp.float32)
                + gb1_ref[0])
    g2 = _leaky(jnp.dot(g1, gW2_ref[0], preferred_element_type=jnp.float32)
                + gb2_ref[0])
    gvec = jnp.dot(g2.astype(jnp.bfloat16), cW_ref[0, D:, :],
                   preferred_element_type=jnp.float32) + cb_ref[0]

    for r in range(NRT):
        ct = _leaky(jnp.dot(h_s[r * RT:(r + 1) * RT, :], cW_ref[0, :D, :],
                            preferred_element_type=jnp.float32) + gvec)
        xn = ct + out_ref[0, r * RT:(r + 1) * RT, :]
        out_ref[0, r * RT:(r + 1) * RT, :] = xn

    @pl.when(lyr == LAYERS - 1)
    def _pool():
        pooled_ref[0] = jnp.sum(out_ref[0], axis=0, keepdims=True) * inv


def _head_body(pooled_ref, oW1_ref, ob1_ref, oW2_ref, ob2_ref, o_ref):
    h = _leaky(jnp.dot(pooled_ref[...], oW1_ref[...],
                       preferred_element_type=jnp.float32) + ob1_ref[...])
    o_ref[...] = _leaky(jnp.dot(h, oW2_ref[...],
                                preferred_element_type=jnp.float32)
                        + ob2_ref[...])


@jax.jit
def kernel(words, action, length, params):
    lp = params["layers"]
    pW1 = jnp.stack([p["pW1"] for p in lp]).astype(jnp.bfloat16)
    pb1 = jnp.stack([p["pb1"] for p in lp])[:, None, :]
    pW2 = jnp.stack([p["pW2"] for p in lp]).astype(jnp.bfloat16)
    pb2 = jnp.stack([p["pb2"] for p in lp])[:, None, :]
    gW1 = jnp.stack([p["gW1"] for p in lp])
    gb1 = jnp.stack([p["gb1"] for p in lp])[:, None, :]
    gW2 = jnp.stack([p["gW2"] for p in lp])
    gb2 = jnp.stack([p["gb2"] for p in lp])[:, None, :]
    cW = jnp.stack([p["cW"] for p in lp]).astype(jnp.bfloat16)
    cb = jnp.stack([p["cb"] for p in lp])[:, None, :]
    emb = jnp.zeros((32, ES), jnp.float32).at[:26].set(params["embed"])
    words3 = words.reshape(B, L, 5)
    invlen = 1.0 / length

    wspec = lambda shape: pl.BlockSpec(
        (1,) + shape, lambda b, l: (l,) + (0,) * len(shape))

    out_buf, pooled = pl.pallas_call(
        _main_body,
        grid=(B, LAYERS),
        in_specs=[
            pl.BlockSpec((1, L, 5), lambda b, l: (b, 0, 0)),
            pl.BlockSpec(memory_space=pltpu.SMEM),
            pl.BlockSpec(memory_space=pltpu.SMEM),
            pl.BlockSpec((32, ES), lambda b, l: (0, 0)),
            wspec((D, H)), wspec((1, H)),
            wspec((H, D)), wspec((1, D)),
            wspec((D, H)), wspec((1, H)),
            wspec((H, D)), wspec((1, D)),
            wspec((2 * D, D)), wspec((1, D)),
        ],
        out_specs=[
            pl.BlockSpec((1, L, D), lambda b, l: (b, 0, 0)),
            pl.BlockSpec((1, 1, D), lambda b, l: (b, 0, 0)),
        ],
        out_shape=[
            jax.ShapeDtypeStruct((B, L, D), jnp.float32),
            jax.ShapeDtypeStruct((B, 1, D), jnp.float32),
        ],
        scratch_shapes=[pltpu.VMEM((L, D), jnp.bfloat16)],
        compiler_params=pltpu.CompilerParams(
            vmem_limit_bytes=128 * 1024 * 1024),
    )(words3, action, invlen, emb,
      pW1, pb1, pW2, pb2, gW1, gb1, gW2, gb2, cW, cb)

    op = params["out"]
    o = pl.pallas_call(
        _head_body,
        out_shape=jax.ShapeDtypeStruct((B, 1), jnp.float32),
    )(pooled.reshape(B, D), op["W1"], op["b1"][None, :],
      op["W2"], op["b2"][None, :])

    return o[:, 0] + params["w"] * jnp.log2(length)


# speedup vs baseline: 1.0352x; 1.0352x over previous
import jax
import jax.numpy as jnp
from jax.experimental import pallas as pl
from jax.experimental.pallas import tpu as pltpu

B = 16
L = 2048
ES = 128
D = ES * 5
H = D * 2
LAYERS = 4
NEG = 0.01
RT = 512
NRT = L // RT


def _leaky(x):
    return jnp.maximum(x, NEG * x)


def _main_body(words_ref, action_ref, invlen_ref, emb_ref,
               pW1_ref, pb1_ref, pW2_ref, pb2_ref,
               gW1_ref, gb1_ref, gW2_ref, gb2_ref,
               cW_ref, cb_ref,
               out_ref, pooled_ref,
               h_s, gsum_s):
    b = pl.program_id(0)
    st = pl.program_id(1)
    inv = invlen_ref[b]

    def token_mlp(xt):
        h1 = _leaky(jnp.dot(xt, pW1_ref[0], preferred_element_type=jnp.float32)
                    + pb1_ref[0])
        return _leaky(jnp.dot(h1.astype(jnp.bfloat16), pW2_ref[0],
                              preferred_element_type=jnp.float32)
                      + pb2_ref[0])

    def gvec_calc():
        grp = gsum_s[0:1, :] * inv
        g1 = _leaky(jnp.dot(grp.astype(jnp.bfloat16), gW1_ref[0],
                            preferred_element_type=jnp.float32) + gb1_ref[0])
        g2 = _leaky(jnp.dot(g1.astype(jnp.bfloat16), gW2_ref[0],
                            preferred_element_type=jnp.float32) + gb2_ref[0])
        return jnp.dot(g2.astype(jnp.bfloat16), cW_ref[0, D:, :],
                       preferred_element_type=jnp.float32) + cb_ref[0]

    @pl.when(st == 0)
    def _first():
        for j in range(5):
            wj = words_ref[0, :, j:j + 1]
            oh = (wj == jax.lax.broadcasted_iota(jnp.int32, (L, 32), 1)
                  ).astype(jnp.float32)
            we = jnp.dot(oh, emb_ref[...], preferred_element_type=jnp.float32)
            aj = action_ref[b, j]
            oha = (aj == jax.lax.broadcasted_iota(jnp.int32, (1, 32), 1)
                   ).astype(jnp.float32)
            ae = jnp.dot(oha, emb_ref[...], preferred_element_type=jnp.float32)
            out_ref[0, :, ES * j:ES * (j + 1)] = we + ae
        gsum = jnp.zeros((1, D), jnp.float32)
        for r in range(NRT):
            sl = pl.ds(r * RT, RT)
            ht = token_mlp(out_ref[0, sl, :].astype(jnp.bfloat16))
            h_s[sl, :] = ht.astype(jnp.bfloat16)
            gsum = gsum + jnp.sum(ht, axis=0, keepdims=True)
        gsum_s[0:1, :] = gsum

    @pl.when((st > 0) & (st < LAYERS))
    def _mid():
        gvec = gvec_calc()
        gsum = jnp.zeros((1, D), jnp.float32)
        for r in range(NRT):
            sl = pl.ds(r * RT, RT)
            ct = _leaky(jnp.dot(h_s[sl, :], cW_ref[0, :D, :],
                                preferred_element_type=jnp.float32) + gvec)
            xn = ct + out_ref[0, sl, :]
            out_ref[0, sl, :] = xn
            ht = token_mlp(xn.astype(jnp.bfloat16))
            h_s[sl, :] = ht.astype(jnp.bfloat16)
            gsum = gsum + jnp.sum(ht, axis=0, keepdims=True)
        gsum_s[0:1, :] = gsum

    @pl.when(st == LAYERS)
    def _last():
        gvec = gvec_calc()
        psum = jnp.zeros((1, D), jnp.float32)
        for r in range(NRT):
            sl = pl.ds(r * RT, RT)
            ct = _leaky(jnp.dot(h_s[sl, :], cW_ref[0, :D, :],
                                preferred_element_type=jnp.float32) + gvec)
            xn = ct + out_ref[0, sl, :]
            psum = psum + jnp.sum(xn, axis=0, keepdims=True)
        pooled_ref[0] = psum * inv


def _head_body(pooled_ref, oW1_ref, ob1_ref, oW2_ref, ob2_ref, o_ref):
    h = _leaky(jnp.dot(pooled_ref[...], oW1_ref[...],
                       preferred_element_type=jnp.float32) + ob1_ref[...])
    o_ref[...] = _leaky(jnp.dot(h, oW2_ref[...],
                                preferred_element_type=jnp.float32)
                        + ob2_ref[...])


@jax.jit
def kernel(words, action, length, params):
    lp = params["layers"]
    pW1 = jnp.stack([p["pW1"] for p in lp]).astype(jnp.bfloat16)
    pb1 = jnp.stack([p["pb1"] for p in lp])[:, None, :]
    pW2 = jnp.stack([p["pW2"] for p in lp]).astype(jnp.bfloat16)
    pb2 = jnp.stack([p["pb2"] for p in lp])[:, None, :]
    gW1 = jnp.stack([p["gW1"] for p in lp]).astype(jnp.bfloat16)
    gb1 = jnp.stack([p["gb1"] for p in lp])[:, None, :]
    gW2 = jnp.stack([p["gW2"] for p in lp]).astype(jnp.bfloat16)
    gb2 = jnp.stack([p["gb2"] for p in lp])[:, None, :]
    cW = jnp.stack([p["cW"] for p in lp]).astype(jnp.bfloat16)
    cb = jnp.stack([p["cb"] for p in lp])[:, None, :]
    emb = jnp.zeros((32, ES), jnp.float32).at[:26].set(params["embed"])
    words3 = words.reshape(B, L, 5)
    invlen = 1.0 / length

    pwspec = lambda shape: pl.BlockSpec(
        (1,) + shape,
        lambda b, s: (jnp.minimum(s, LAYERS - 1),) + (0,) * len(shape))
    cwspec = lambda shape: pl.BlockSpec(
        (1,) + shape,
        lambda b, s: (jnp.maximum(s - 1, 0),) + (0,) * len(shape))

    out_buf, pooled = pl.pallas_call(
        _main_body,
        grid=(B, LAYERS + 1),
        in_specs=[
            pl.BlockSpec((1, L, 5), lambda b, s: (b, 0, 0)),
            pl.BlockSpec(memory_space=pltpu.SMEM),
            pl.BlockSpec(memory_space=pltpu.SMEM),
            pl.BlockSpec((32, ES), lambda b, s: (0, 0)),
            pwspec((D, H)), pwspec((1, H)),
            pwspec((H, D)), pwspec((1, D)),
            cwspec((D, H)), cwspec((1, H)),
            cwspec((H, D)), cwspec((1, D)),
            cwspec((2 * D, D)), cwspec((1, D)),
        ],
        out_specs=[
            pl.BlockSpec((1, L, D), lambda b, s: (b, 0, 0)),
            pl.BlockSpec((1, 1, D), lambda b, s: (b, 0, 0)),
        ],
        out_shape=[
            jax.ShapeDtypeStruct((B, L, D), jnp.float32),
            jax.ShapeDtypeStruct((B, 1, D), jnp.float32),
        ],
        scratch_shapes=[pltpu.VMEM((L, D), jnp.bfloat16),
                        pltpu.VMEM((1, D), jnp.float32)],
        compiler_params=pltpu.CompilerParams(
            vmem_limit_bytes=128 * 1024 * 1024),
    )(words3, action, invlen, emb,
      pW1, pb1, pW2, pb2, gW1, gb1, gW2, gb2, cW, cb)

    op = params["out"]
    o = pl.pallas_call(
        _head_body,
        out_shape=jax.ShapeDtypeStruct((B, 1), jnp.float32),
    )(pooled.reshape(B, D), op["W1"], op["b1"][None, :],
      op["W2"], op["b2"][None, :])

    return o[:, 0] + params["w"] * jnp.log2(length)
